# baseline (speedup 1.0000x reference)
import functools

import jax
import jax.numpy as jnp
from jax.experimental import pallas as pl

TOP_K = 8
NUM_EXPERTS = 64
TILE_M = 1024


def _gate_kernel(x_ref, wt_ref, sel_ref, fus_ref, idx_ref, val_ref):
    scores = jnp.dot(x_ref[...], wt_ref[...], preferred_element_type=jnp.float32)
    sel = scores[:, :NUM_EXPERTS]
    sel_ref[...] = sel
    fus_ref[...] = scores[:, NUM_EXPERTS:]

    work = sel.T
    expert = jax.lax.broadcasted_iota(jnp.int32, work.shape, 0).astype(jnp.float32)
    neg_inf = jnp.float32(-jnp.inf)
    vals = []
    idxs = []
    for k in range(TOP_K):
        m = jnp.max(work, axis=0, keepdims=True)
        hit = work == m
        idx = jnp.min(jnp.where(hit, expert, jnp.float32(NUM_EXPERTS)), axis=0,
                      keepdims=True)
        vals.append(m)
        idxs.append(idx)
        work = jnp.where(expert == idx, neg_inf, work)
    val_ref[...] = jnp.concatenate(vals, axis=0).T
    idx_ref[...] = jnp.concatenate(idxs, axis=0).T.astype(jnp.int32)


@functools.partial(jax.jit, static_argnums=())
def kernel(x, W_sel, W_fus):
    m, d = x.shape
    wt = jnp.concatenate([W_sel, W_fus], axis=0).T
    grid = (m // TILE_M,)
    sel, fus, idx, val = pl.pallas_call(
        _gate_kernel,
        grid=grid,
        in_specs=[
            pl.BlockSpec((TILE_M, d), lambda i: (i, 0)),
            pl.BlockSpec((d, 2 * NUM_EXPERTS), lambda i: (0, 0)),
        ],
        out_specs=[
            pl.BlockSpec((TILE_M, NUM_EXPERTS), lambda i: (i, 0)),
            pl.BlockSpec((TILE_M, NUM_EXPERTS), lambda i: (i, 0)),
            pl.BlockSpec((TILE_M, TOP_K), lambda i: (i, 0)),
            pl.BlockSpec((TILE_M, TOP_K), lambda i: (i, 0)),
        ],
        out_shape=[
            jax.ShapeDtypeStruct((m, NUM_EXPERTS), jnp.float32),
            jax.ShapeDtypeStruct((m, NUM_EXPERTS), jnp.float32),
            jax.ShapeDtypeStruct((m, TOP_K), jnp.int32),
            jax.ShapeDtypeStruct((m, TOP_K), jnp.float32),
        ],
    )(x, wt)
    return (sel, fus, idx, val)


# speedup vs baseline: 1.1452x; 1.1452x over previous
import functools

import jax
import jax.numpy as jnp
from jax.experimental import pallas as pl

TOP_K = 8
NUM_EXPERTS = 64
TILE_M = 1024


def _gate_kernel(x_ref, wt_ref, sel_ref, fus_ref, idx_ref, val_ref):
    scores = jnp.dot(x_ref[...], wt_ref[...], preferred_element_type=jnp.float32)
    sel = scores[:, :NUM_EXPERTS]
    sel_ref[...] = sel
    fus_ref[...] = scores[:, NUM_EXPERTS:]

    work = sel.T
    expert = jax.lax.broadcasted_iota(jnp.int32, work.shape, 0).astype(jnp.float32)
    neg_inf = jnp.float32(-jnp.inf)
    vals = []
    idxs = []
    for k in range(TOP_K):
        m = jnp.max(work, axis=0, keepdims=True)
        hit = work == m
        idx = jnp.min(jnp.where(hit, expert, jnp.float32(NUM_EXPERTS)), axis=0,
                      keepdims=True)
        vals.append(m)
        idxs.append(idx)
        work = jnp.where(expert == idx, neg_inf, work)
    val_ref[...] = jnp.concatenate(vals, axis=0)
    idx_ref[...] = jnp.concatenate(idxs, axis=0).astype(jnp.int32)


@functools.partial(jax.jit, static_argnums=())
def kernel(x, W_sel, W_fus):
    m, d = x.shape
    wt = jnp.concatenate([W_sel, W_fus], axis=0).T
    grid = (m // TILE_M,)
    sel, fus, idx_t, val_t = pl.pallas_call(
        _gate_kernel,
        grid=grid,
        in_specs=[
            pl.BlockSpec((TILE_M, d), lambda i: (i, 0)),
            pl.BlockSpec((d, 2 * NUM_EXPERTS), lambda i: (0, 0)),
        ],
        out_specs=[
            pl.BlockSpec((TILE_M, NUM_EXPERTS), lambda i: (i, 0)),
            pl.BlockSpec((TILE_M, NUM_EXPERTS), lambda i: (i, 0)),
            pl.BlockSpec((TOP_K, TILE_M), lambda i: (0, i)),
            pl.BlockSpec((TOP_K, TILE_M), lambda i: (0, i)),
        ],
        out_shape=[
            jax.ShapeDtypeStruct((m, NUM_EXPERTS), jnp.float32),
            jax.ShapeDtypeStruct((m, NUM_EXPERTS), jnp.float32),
            jax.ShapeDtypeStruct((TOP_K, m), jnp.int32),
            jax.ShapeDtypeStruct((TOP_K, m), jnp.float32),
        ],
    )(x, wt)
    return (sel, fus, idx_t.T, val_t.T)


# speedup vs baseline: 1.1771x; 1.0279x over previous
import functools

import jax
import jax.numpy as jnp
from jax.experimental import pallas as pl

TOP_K = 8
NUM_EXPERTS = 64
TILE_M = 1024


def _gate_kernel(x_ref, wt_ref, sel_ref, fus_ref, idx_ref, val_ref):
    scores = jax.lax.dot_general(
        x_ref[...], wt_ref[...], (((1,), (1,)), ((), ())),
        preferred_element_type=jnp.float32)
    sel = scores[:, :NUM_EXPERTS]
    sel_ref[...] = sel
    fus_ref[...] = scores[:, NUM_EXPERTS:]

    work = sel.T
    expert = jax.lax.broadcasted_iota(jnp.int32, work.shape, 0).astype(jnp.float32)
    neg_inf = jnp.float32(-jnp.inf)
    vals = []
    idxs = []
    for k in range(TOP_K):
        m = jnp.max(work, axis=0, keepdims=True)
        hit = work == m
        idx = jnp.min(jnp.where(hit, expert, jnp.float32(NUM_EXPERTS)), axis=0,
                      keepdims=True)
        vals.append(m)
        idxs.append(idx)
        work = jnp.where(expert == idx, neg_inf, work)
    val_ref[...] = jnp.concatenate(vals, axis=0)
    idx_ref[...] = jnp.concatenate(idxs, axis=0).astype(jnp.int32)


@functools.partial(jax.jit, static_argnums=())
def kernel(x, W_sel, W_fus):
    m, d = x.shape
    wt = jnp.concatenate([W_sel, W_fus], axis=0)
    grid = (m // TILE_M,)
    sel, fus, idx_t, val_t = pl.pallas_call(
        _gate_kernel,
        grid=grid,
        in_specs=[
            pl.BlockSpec((TILE_M, d), lambda i: (i, 0)),
            pl.BlockSpec((2 * NUM_EXPERTS, d), lambda i: (0, 0)),
        ],
        out_specs=[
            pl.BlockSpec((TILE_M, NUM_EXPERTS), lambda i: (i, 0)),
            pl.BlockSpec((TILE_M, NUM_EXPERTS), lambda i: (i, 0)),
            pl.BlockSpec((TOP_K, TILE_M), lambda i: (0, i)),
            pl.BlockSpec((TOP_K, TILE_M), lambda i: (0, i)),
        ],
        out_shape=[
            jax.ShapeDtypeStruct((m, NUM_EXPERTS), jnp.float32),
            jax.ShapeDtypeStruct((m, NUM_EXPERTS), jnp.float32),
            jax.ShapeDtypeStruct((TOP_K, m), jnp.int32),
            jax.ShapeDtypeStruct((TOP_K, m), jnp.float32),
        ],
    )(x, wt)
    return (sel, fus, idx_t.T, val_t.T)
